# Initial kernel scaffold; baseline (speedup 1.0000x reference)
import jax
import jax.numpy as jnp
from jax.experimental import pallas as pl


def kernel(xyz, new_xyz, features):
    raise NotImplementedError("write your pallas kernel here")


# speedup vs baseline: 40.3839x; 40.3839x over previous
import functools

import jax
import jax.numpy as jnp
from jax import lax
from jax.experimental import pallas as pl
from jax.experimental.pallas import tpu as pltpu
from jax.experimental.pallas import tpu_sc as plsc

RADIUS2 = 0.2 * 0.2
NSAMPLE = 32

B, N, NPOINT, C = 4, 16384, 1024, 64
NCHUNK = N // 16

try:
    _info = plsc.get_sparse_core_info()
    NC, NS = _info.num_cores, _info.num_subcores
except Exception:
    NC, NS = 2, 16
NUM_CORES = NC
NUM_SUBCORES = NS
NW = NC * NS
TPB = NW // B
CPT = NPOINT // TPB
FPT = C // TPB

_mesh = plsc.VectorSubcoreMesh(core_axis_name="c", subcore_axis_name="s",
                               num_cores=NC, num_subcores=NS)


@functools.partial(
    pl.kernel,
    out_type=jax.ShapeDtypeStruct((B, NPOINT, NSAMPLE), jnp.int32),
    mesh=_mesh,
    compiler_params=pltpu.CompilerParams(needs_layout_passes=False),
    scratch_types=[
        pltpu.VMEM((N,), jnp.float32),
        pltpu.VMEM((N,), jnp.float32),
        pltpu.VMEM((N,), jnp.float32),
        pltpu.VMEM((N,), jnp.float32),
        pltpu.VMEM((CPT,), jnp.float32),
        pltpu.VMEM((CPT,), jnp.float32),
        pltpu.VMEM((CPT,), jnp.float32),
        pltpu.VMEM((CPT,), jnp.float32),
        pltpu.VMEM((64,), jnp.int32),
        pltpu.VMEM((CPT, NSAMPLE), jnp.int32),
    ],
)
def _ball_query_sc(xb_hbm, yb_hbm, zb_hbm, pp_hbm, cx_hbm, cy_hbm, cz_hbm,
                   cc_hbm, idx_hbm, xv, yv, zv, ppv, cxv, cyv, czv, ccv,
                   hits, idxout):
    wid = lax.axis_index("s") * NC + lax.axis_index("c")
    b = wid // TPB
    seg = wid % TPB
    pltpu.sync_copy(xb_hbm.at[b], xv)
    pltpu.sync_copy(yb_hbm.at[b], yv)
    pltpu.sync_copy(zb_hbm.at[b], zv)
    pltpu.sync_copy(pp_hbm.at[b], ppv)
    pltpu.sync_copy(cx_hbm.at[b, pl.ds(seg * CPT, CPT)], cxv)
    pltpu.sync_copy(cy_hbm.at[b, pl.ds(seg * CPT, CPT)], cyv)
    pltpu.sync_copy(cz_hbm.at[b, pl.ds(seg * CPT, CPT)], czv)
    pltpu.sync_copy(cc_hbm.at[b, pl.ds(seg * CPT, CPT)], ccv)

    lanes = lax.iota(jnp.int32, 16)
    r2 = jnp.full((16,), RADIUS2, jnp.float32)

    def per_centroid(p, _):
        pv = jnp.full((16,), p, jnp.int32)
        cx = plsc.load_gather(cxv, [pv])
        cy = plsc.load_gather(cyv, [pv])
        cz = plsc.load_gather(czv, [pv])
        cc = plsc.load_gather(ccv, [pv])

        def cond(carry):
            i, cnt = carry
            return (cnt < NSAMPLE) & (i < NCHUNK)

        def body(carry):
            i, cnt = carry
            base = i * 16
            px = xv[pl.ds(base, 16)]
            py = yv[pl.ds(base, 16)]
            pz = zv[pl.ds(base, 16)]
            pp = ppv[pl.ds(base, 16)]
            dot = (cx * px + cy * py) + cz * pz
            d2 = (cc + pp) - (dot + dot)
            m = d2 <= r2
            plsc.store_compressed(hits.at[pl.ds(cnt, 16)], base + lanes,
                                  mask=m)
            cnt = cnt + jnp.sum(m.astype(jnp.int32))
            return i + 1, cnt

        _, cnt = lax.while_loop(cond, body, (jnp.int32(0), jnp.int32(0)))

        pad = plsc.load_gather(hits, [jnp.zeros((16,), jnp.int32)])
        h0 = hits[pl.ds(0, 16)]
        h1 = hits[pl.ds(16, 16)]
        idxout[p, pl.ds(0, 16)] = jnp.where(lanes < cnt, h0, pad)
        idxout[p, pl.ds(16, 16)] = jnp.where(lanes + 16 < cnt, h1, pad)
        return ()

    lax.fori_loop(0, CPT, per_centroid, ())
    pltpu.sync_copy(idxout, idx_hbm.at[b, pl.ds(seg * CPT, CPT)])


@functools.partial(
    pl.kernel,
    out_type=jax.ShapeDtypeStruct((B, 6 + C, NPOINT * NSAMPLE), jnp.float32),
    mesh=_mesh,
    compiler_params=pltpu.CompilerParams(needs_layout_passes=False),
    scratch_types=[
        pltpu.VMEM((NPOINT * NSAMPLE,), jnp.int32),
        pltpu.VMEM((N,), jnp.float32),
        pltpu.VMEM((NPOINT,), jnp.float32),
        pltpu.VMEM((NPOINT * NSAMPLE,), jnp.float32),
    ],
)
def _group_sc(idx_hbm, xyzt_hbm, cent_hbm, feat_hbm, out_hbm,
              idxv, row, cents, outbuf):
    wid = lax.axis_index("s") * NC + lax.axis_index("c")
    b = wid // TPB
    r = wid % TPB
    pltpu.sync_copy(idx_hbm.at[b], idxv)

    def gather_row():
        def gstep(j, _):
            iv = idxv[pl.ds(j * 16, 16)]
            outbuf[pl.ds(j * 16, 16)] = plsc.load_gather(row, [iv])
            return ()
        lax.fori_loop(0, NPOINT * NSAMPLE // 16, gstep, ())

    for k in range(FPT):
        ch = r + k * TPB
        pltpu.sync_copy(feat_hbm.at[b, ch], row)
        gather_row()
        pltpu.sync_copy(outbuf, out_hbm.at[b, 6 + ch])

    @pl.when(r < 3)
    def _():
        d = r
        pltpu.sync_copy(xyzt_hbm.at[b, d], row)
        pltpu.sync_copy(cent_hbm.at[b, d], cents)

        def cstep(p, _):
            pv = jnp.full((16,), p, jnp.int32)
            cb = plsc.load_gather(cents, [pv])
            iv0 = idxv[pl.ds(p * NSAMPLE, 16)]
            iv1 = idxv[pl.ds(p * NSAMPLE + 16, 16)]
            outbuf[pl.ds(p * NSAMPLE, 16)] = plsc.load_gather(row, [iv0]) - cb
            outbuf[pl.ds(p * NSAMPLE + 16, 16)] = (
                plsc.load_gather(row, [iv1]) - cb)
            return ()

        lax.fori_loop(0, NPOINT, cstep, ())
        pltpu.sync_copy(outbuf, out_hbm.at[b, d])
        pltpu.sync_copy(outbuf, out_hbm.at[b, 3 + d])


def _round_bf16(x):
    u = lax.bitcast_convert_type(x, jnp.uint32)
    r = u + jnp.uint32(0x7FFF) + ((u >> 16) & jnp.uint32(1))
    return lax.bitcast_convert_type(r & jnp.uint32(0xFFFF0000), jnp.float32)


def kernel(xyz, new_xyz, features):
    xb = _round_bf16(xyz)
    nb = _round_bf16(new_xyz)
    cc = jnp.sum(new_xyz * new_xyz, axis=-1)
    pp = jnp.sum(xyz * xyz, axis=-1)
    xbx, xby, xbz = xb[..., 0], xb[..., 1], xb[..., 2]
    nbx, nby, nbz = nb[..., 0], nb[..., 1], nb[..., 2]
    idx = _ball_query_sc(xbx, xby, xbz, pp, nbx, nby, nbz, cc)
    xyzt = jnp.transpose(xyz, (0, 2, 1))
    cent = jnp.transpose(new_xyz, (0, 2, 1))
    idx2 = idx.reshape(B, NPOINT * NSAMPLE)
    out = _group_sc(idx2, xyzt, cent, features)
    return out.reshape(B, 6 + C, NPOINT, NSAMPLE)


# speedup vs baseline: 58.6286x; 1.4518x over previous
import functools

import jax
import jax.numpy as jnp
from jax import lax
from jax.experimental import pallas as pl
from jax.experimental.pallas import tpu as pltpu
from jax.experimental.pallas import tpu_sc as plsc

RADIUS2 = 0.2 * 0.2
NSAMPLE = 32

B, N, NPOINT, C = 4, 16384, 1024, 64
NCHUNK = N // 16

try:
    _info = plsc.get_sparse_core_info()
    NC, NS = _info.num_cores, _info.num_subcores
except Exception:
    NC, NS = 2, 16
NUM_CORES = NC
NUM_SUBCORES = NS
NW = NC * NS
TPB = NW // B
CPT = NPOINT // TPB
FPT = C // TPB

_mesh = plsc.VectorSubcoreMesh(core_axis_name="c", subcore_axis_name="s",
                               num_cores=NC, num_subcores=NS)


@functools.partial(
    pl.kernel,
    out_type=jax.ShapeDtypeStruct((B, NPOINT, NSAMPLE), jnp.int32),
    mesh=_mesh,
    compiler_params=pltpu.CompilerParams(needs_layout_passes=False),
    scratch_types=[
        pltpu.VMEM((N,), jnp.float32),
        pltpu.VMEM((N,), jnp.float32),
        pltpu.VMEM((N,), jnp.float32),
        pltpu.VMEM((N,), jnp.float32),
        pltpu.VMEM((CPT,), jnp.float32),
        pltpu.VMEM((CPT,), jnp.float32),
        pltpu.VMEM((CPT,), jnp.float32),
        pltpu.VMEM((CPT,), jnp.float32),
        pltpu.VMEM((64,), jnp.int32),
        pltpu.VMEM((CPT, NSAMPLE), jnp.int32),
    ],
)
def _ball_query_sc(xb_hbm, yb_hbm, zb_hbm, pp_hbm, cx_hbm, cy_hbm, cz_hbm,
                   cc_hbm, idx_hbm, xv, yv, zv, ppv, cxv, cyv, czv, ccv,
                   hits, idxout):
    wid = lax.axis_index("s") * NC + lax.axis_index("c")
    b = wid // TPB
    seg = wid % TPB
    pltpu.sync_copy(xb_hbm.at[b], xv)
    pltpu.sync_copy(yb_hbm.at[b], yv)
    pltpu.sync_copy(zb_hbm.at[b], zv)
    pltpu.sync_copy(pp_hbm.at[b], ppv)
    pltpu.sync_copy(cx_hbm.at[b, pl.ds(seg * CPT, CPT)], cxv)
    pltpu.sync_copy(cy_hbm.at[b, pl.ds(seg * CPT, CPT)], cyv)
    pltpu.sync_copy(cz_hbm.at[b, pl.ds(seg * CPT, CPT)], czv)
    pltpu.sync_copy(cc_hbm.at[b, pl.ds(seg * CPT, CPT)], ccv)

    lanes = lax.iota(jnp.int32, 16)
    r2 = jnp.full((16,), RADIUS2, jnp.float32)

    def per_centroid(p, _):
        pv = jnp.full((16,), p, jnp.int32)
        cx = plsc.load_gather(cxv, [pv])
        cy = plsc.load_gather(cyv, [pv])
        cz = plsc.load_gather(czv, [pv])
        cc = plsc.load_gather(ccv, [pv])

        def cond(carry):
            i, cnt = carry
            return (cnt < NSAMPLE) & (i < NCHUNK // 2)

        def chunk(base, cnt):
            px = xv[pl.ds(base, 16)]
            py = yv[pl.ds(base, 16)]
            pz = zv[pl.ds(base, 16)]
            pp = ppv[pl.ds(base, 16)]
            dot = (cx * px + cy * py) + cz * pz
            d2 = (cc + pp) - (dot + dot)
            m = d2 <= r2
            plsc.store_compressed(hits.at[pl.ds(cnt, 16)], base + lanes,
                                  mask=m)
            return cnt + plsc.all_reduce_population_count(m)[0]

        def body(carry):
            i, cnt = carry
            base = i * 32
            cnt = chunk(base, cnt)
            cnt = chunk(base + 16, cnt)
            return i + 1, cnt

        _, cnt = lax.while_loop(cond, body, (jnp.int32(0), jnp.int32(0)))

        pad = plsc.load_gather(hits, [jnp.zeros((16,), jnp.int32)])
        h0 = hits[pl.ds(0, 16)]
        h1 = hits[pl.ds(16, 16)]
        idxout[p, pl.ds(0, 16)] = jnp.where(lanes < cnt, h0, pad)
        idxout[p, pl.ds(16, 16)] = jnp.where(lanes + 16 < cnt, h1, pad)
        return ()

    lax.fori_loop(0, CPT, per_centroid, ())
    pltpu.sync_copy(idxout, idx_hbm.at[b, pl.ds(seg * CPT, CPT)])


@functools.partial(
    pl.kernel,
    out_type=jax.ShapeDtypeStruct((B, 6 + C, NPOINT * NSAMPLE), jnp.float32),
    mesh=_mesh,
    compiler_params=pltpu.CompilerParams(needs_layout_passes=False),
    scratch_types=[
        pltpu.VMEM((NPOINT * NSAMPLE,), jnp.int32),
        pltpu.VMEM((N,), jnp.float32),
        pltpu.VMEM((NPOINT,), jnp.float32),
        pltpu.VMEM((NPOINT * NSAMPLE,), jnp.float32),
    ],
)
def _group_sc(idx_hbm, xyzt_hbm, cent_hbm, feat_hbm, out_hbm,
              idxv, row, cents, outbuf):
    wid = lax.axis_index("s") * NC + lax.axis_index("c")
    b = wid // TPB
    r = wid % TPB
    pltpu.sync_copy(idx_hbm.at[b], idxv)

    def gather_row():
        def gstep(j):
            iv = idxv[pl.ds(j * 16, 16)]
            outbuf[pl.ds(j * 16, 16)] = plsc.load_gather(row, [iv])
        plsc.parallel_loop(0, NPOINT * NSAMPLE // 16, 1, unroll=8)(gstep)

    for k in range(FPT):
        ch = r + k * TPB
        pltpu.sync_copy(feat_hbm.at[b, ch], row)
        gather_row()
        pltpu.sync_copy(outbuf, out_hbm.at[b, 6 + ch])

    @pl.when(r < 3)
    def _():
        d = r
        pltpu.sync_copy(xyzt_hbm.at[b, d], row)
        pltpu.sync_copy(cent_hbm.at[b, d], cents)

        def cstep(p, _):
            pv = jnp.full((16,), p, jnp.int32)
            cb = plsc.load_gather(cents, [pv])
            iv0 = idxv[pl.ds(p * NSAMPLE, 16)]
            iv1 = idxv[pl.ds(p * NSAMPLE + 16, 16)]
            outbuf[pl.ds(p * NSAMPLE, 16)] = plsc.load_gather(row, [iv0]) - cb
            outbuf[pl.ds(p * NSAMPLE + 16, 16)] = (
                plsc.load_gather(row, [iv1]) - cb)
            return ()

        lax.fori_loop(0, NPOINT, cstep, ())
        pltpu.sync_copy(outbuf, out_hbm.at[b, d])
        pltpu.sync_copy(outbuf, out_hbm.at[b, 3 + d])


def _round_bf16(x):
    u = lax.bitcast_convert_type(x, jnp.uint32)
    r = u + jnp.uint32(0x7FFF) + ((u >> 16) & jnp.uint32(1))
    return lax.bitcast_convert_type(r & jnp.uint32(0xFFFF0000), jnp.float32)


def kernel(xyz, new_xyz, features):
    xb = _round_bf16(xyz)
    nb = _round_bf16(new_xyz)
    cc = jnp.sum(new_xyz * new_xyz, axis=-1)
    pp = jnp.sum(xyz * xyz, axis=-1)
    xbx, xby, xbz = xb[..., 0], xb[..., 1], xb[..., 2]
    nbx, nby, nbz = nb[..., 0], nb[..., 1], nb[..., 2]
    idx = _ball_query_sc(xbx, xby, xbz, pp, nbx, nby, nbz, cc)
    xyzt = jnp.transpose(xyz, (0, 2, 1))
    cent = jnp.transpose(new_xyz, (0, 2, 1))
    idx2 = idx.reshape(B, NPOINT * NSAMPLE)
    out = _group_sc(idx2, xyzt, cent, features)
    return out.reshape(B, 6 + C, NPOINT, NSAMPLE)


# speedup vs baseline: 65.2992x; 1.1138x over previous
import functools

import jax
import jax.numpy as jnp
from jax import lax
from jax.experimental import pallas as pl
from jax.experimental.pallas import tpu as pltpu
from jax.experimental.pallas import tpu_sc as plsc

RADIUS2 = 0.2 * 0.2
NSAMPLE = 32

B, N, NPOINT, C = 4, 16384, 1024, 64
NCHUNK = N // 16

try:
    _info = plsc.get_sparse_core_info()
    NC, NS = _info.num_cores, _info.num_subcores
except Exception:
    NC, NS = 2, 16
NUM_CORES = NC
NUM_SUBCORES = NS
NW = NC * NS
TPB = NW // B
CPT = NPOINT // TPB
FPT = C // TPB

_mesh = plsc.VectorSubcoreMesh(core_axis_name="c", subcore_axis_name="s",
                               num_cores=NC, num_subcores=NS)


@functools.partial(
    pl.kernel,
    out_type=jax.ShapeDtypeStruct((B, NPOINT, NSAMPLE), jnp.int32),
    mesh=_mesh,
    compiler_params=pltpu.CompilerParams(needs_layout_passes=False),
    scratch_types=[
        pltpu.VMEM((N,), jnp.float32),
        pltpu.VMEM((N,), jnp.float32),
        pltpu.VMEM((N,), jnp.float32),
        pltpu.VMEM((N,), jnp.float32),
        pltpu.VMEM((CPT,), jnp.float32),
        pltpu.VMEM((CPT,), jnp.float32),
        pltpu.VMEM((CPT,), jnp.float32),
        pltpu.VMEM((CPT,), jnp.float32),
        pltpu.VMEM((128,), jnp.int32),
        pltpu.VMEM((CPT, NSAMPLE), jnp.int32),
    ],
)
def _ball_query_sc(xb_hbm, yb_hbm, zb_hbm, pp_hbm, cx_hbm, cy_hbm, cz_hbm,
                   cc_hbm, idx_hbm, xv, yv, zv, ppv, cxv, cyv, czv, ccv,
                   hits, idxout):
    wid = lax.axis_index("s") * NC + lax.axis_index("c")
    b = wid // TPB
    seg = wid % TPB
    pltpu.sync_copy(xb_hbm.at[b], xv)
    pltpu.sync_copy(yb_hbm.at[b], yv)
    pltpu.sync_copy(zb_hbm.at[b], zv)
    pltpu.sync_copy(pp_hbm.at[b], ppv)
    pltpu.sync_copy(cx_hbm.at[b, pl.ds(seg * CPT, CPT)], cxv)
    pltpu.sync_copy(cy_hbm.at[b, pl.ds(seg * CPT, CPT)], cyv)
    pltpu.sync_copy(cz_hbm.at[b, pl.ds(seg * CPT, CPT)], czv)
    pltpu.sync_copy(cc_hbm.at[b, pl.ds(seg * CPT, CPT)], ccv)

    lanes = lax.iota(jnp.int32, 16)
    r2 = jnp.full((16,), RADIUS2, jnp.float32)

    def per_centroid(p, _):
        pv = jnp.full((16,), p, jnp.int32)
        cx = plsc.load_gather(cxv, [pv])
        cy = plsc.load_gather(cyv, [pv])
        cz = plsc.load_gather(czv, [pv])
        cc = plsc.load_gather(ccv, [pv])

        def cond(carry):
            i, cnt = carry
            return (cnt < NSAMPLE) & (i < NCHUNK // 4)

        def chunk(base, cnt):
            px = xv[pl.ds(base, 16)]
            py = yv[pl.ds(base, 16)]
            pz = zv[pl.ds(base, 16)]
            pp = ppv[pl.ds(base, 16)]
            dot = (cx * px + cy * py) + cz * pz
            d2 = (cc + pp) - (dot + dot)
            m = d2 <= r2
            plsc.store_compressed(hits.at[pl.ds(cnt, 16)], base + lanes,
                                  mask=m)
            return cnt + plsc.all_reduce_population_count(m)[0]

        def body(carry):
            i, cnt = carry
            base = i * 64
            cnt = chunk(base, cnt)
            cnt = chunk(base + 16, cnt)
            cnt = chunk(base + 32, cnt)
            cnt = chunk(base + 48, cnt)
            return i + 1, cnt

        _, cnt = lax.while_loop(cond, body, (jnp.int32(0), jnp.int32(0)))

        pad = plsc.load_gather(hits, [jnp.zeros((16,), jnp.int32)])
        h0 = hits[pl.ds(0, 16)]
        h1 = hits[pl.ds(16, 16)]
        idxout[p, pl.ds(0, 16)] = jnp.where(lanes < cnt, h0, pad)
        idxout[p, pl.ds(16, 16)] = jnp.where(lanes + 16 < cnt, h1, pad)
        return ()

    lax.fori_loop(0, CPT, per_centroid, ())
    pltpu.sync_copy(idxout, idx_hbm.at[b, pl.ds(seg * CPT, CPT)])


@functools.partial(
    pl.kernel,
    out_type=jax.ShapeDtypeStruct((B, 6 + C, NPOINT * NSAMPLE), jnp.float32),
    mesh=_mesh,
    compiler_params=pltpu.CompilerParams(needs_layout_passes=False),
    scratch_types=[
        pltpu.VMEM((NPOINT * NSAMPLE,), jnp.int32),
        pltpu.VMEM((N,), jnp.float32),
        pltpu.VMEM((NPOINT,), jnp.float32),
        pltpu.VMEM((NPOINT * NSAMPLE,), jnp.float32),
    ],
)
def _group_sc(idx_hbm, xyzt_hbm, cent_hbm, feat_hbm, out_hbm,
              idxv, row, cents, outbuf):
    wid = lax.axis_index("s") * NC + lax.axis_index("c")
    b = wid // TPB
    r = wid % TPB
    pltpu.sync_copy(idx_hbm.at[b], idxv)

    def gather_row():
        def gstep(j):
            iv = idxv[pl.ds(j * 16, 16)]
            outbuf[pl.ds(j * 16, 16)] = plsc.load_gather(row, [iv])
        plsc.parallel_loop(0, NPOINT * NSAMPLE // 16, 1, unroll=8)(gstep)

    for k in range(FPT):
        ch = r + k * TPB
        pltpu.sync_copy(feat_hbm.at[b, ch], row)
        gather_row()
        pltpu.sync_copy(outbuf, out_hbm.at[b, 6 + ch])

    @pl.when(r < 3)
    def _():
        d = r
        pltpu.sync_copy(xyzt_hbm.at[b, d], row)
        pltpu.sync_copy(cent_hbm.at[b, d], cents)

        def cstep(p):
            pv = jnp.full((16,), p, jnp.int32)
            cb = plsc.load_gather(cents, [pv])
            iv0 = idxv[pl.ds(p * NSAMPLE, 16)]
            iv1 = idxv[pl.ds(p * NSAMPLE + 16, 16)]
            outbuf[pl.ds(p * NSAMPLE, 16)] = plsc.load_gather(row, [iv0]) - cb
            outbuf[pl.ds(p * NSAMPLE + 16, 16)] = (
                plsc.load_gather(row, [iv1]) - cb)

        plsc.parallel_loop(0, NPOINT, 1, unroll=4)(cstep)
        pltpu.sync_copy(outbuf, out_hbm.at[b, d])
        pltpu.sync_copy(outbuf, out_hbm.at[b, 3 + d])


def _round_bf16(x):
    u = lax.bitcast_convert_type(x, jnp.uint32)
    r = u + jnp.uint32(0x7FFF) + ((u >> 16) & jnp.uint32(1))
    return lax.bitcast_convert_type(r & jnp.uint32(0xFFFF0000), jnp.float32)


def kernel(xyz, new_xyz, features):
    xb = _round_bf16(xyz)
    nb = _round_bf16(new_xyz)
    cc = jnp.sum(new_xyz * new_xyz, axis=-1)
    pp = jnp.sum(xyz * xyz, axis=-1)
    xbx, xby, xbz = xb[..., 0], xb[..., 1], xb[..., 2]
    nbx, nby, nbz = nb[..., 0], nb[..., 1], nb[..., 2]
    idx = _ball_query_sc(xbx, xby, xbz, pp, nbx, nby, nbz, cc)
    xyzt = jnp.transpose(xyz, (0, 2, 1))
    cent = jnp.transpose(new_xyz, (0, 2, 1))
    idx2 = idx.reshape(B, NPOINT * NSAMPLE)
    out = _group_sc(idx2, xyzt, cent, features)
    return out.reshape(B, 6 + C, NPOINT, NSAMPLE)
